# baseline (speedup 1.0000x reference)
import functools

import jax
import jax.numpy as jnp
from jax import lax
from jax.experimental import pallas as pl
from jax.experimental.pallas import tpu as pltpu
from jax.experimental.pallas import tpu_sc as plsc

B = 8
N0 = 128 * 128
N1 = 64 * 64
K0 = int(0.75 * N0)
K1 = int(max(0.75, 0.9) * N1)

HB = 1024
HOFF = 7168


def _make_sc_body(n, k_sel):
    nch = n // 16

    def body(bits_hbm, out_hbm, data_v, hist_v, tot_v, comp_v):
        core = lax.axis_index("c")
        sub = lax.axis_index("s")
        wid = sub * 2 + core

        @pl.when(wid < 8)
        def _():
            lane = jnp.arange(16, dtype=jnp.int32)
            ones = jnp.ones((16,), jnp.int32)
            zeros16 = jnp.zeros((16,), jnp.int32)
            lane_hb = lane * HB
            k = k_sel

            pltpu.sync_copy(bits_hbm.at[wid], data_v)

            def zero_hist(i, c):
                for u in range(4):
                    hist_v[pl.ds(i * 64 + u * 16, 16)] = zeros16
                return c

            lax.fori_loop(0, (16 * HB) // 64, zero_hist, 0)

            def hpass(i, c):
                for u in range(4):
                    v = data_v[pl.ds((i * 4 + u) * 16, 16)]
                    b = jnp.maximum(lax.shift_right_logical(v, 17) - HOFF, 0)
                    plsc.addupdate_scatter(hist_v, [lane_hb + b], ones)
                return c

            lax.fori_loop(0, nch // 4, hpass, 0)

            def collapse(i, c):
                acc = hist_v[pl.ds(i * 16, 16)]
                for l in range(1, 16):
                    acc = acc + hist_v[pl.ds(l * HB + i * 16, 16)]
                tot_v[pl.ds(i * 16, 16)] = acc
                return c

            lax.fori_loop(0, HB // 16, collapse, 0)

            def scan1(i, carry):
                csum, c_star, csum_b = carry
                c = HB // 16 - 1 - i
                v = tot_v[pl.ds(c * 16, 16)]
                s = jnp.sum(v, axis=0)
                found = jnp.logical_and(csum < k, csum + s >= k)
                c_star = jnp.where(found, c, c_star)
                csum_b = jnp.where(found, csum, csum_b)
                return csum + s, c_star, csum_b

            _, c_star, csum_b = lax.fori_loop(0, HB // 16, scan1, (0, 0, 0))

            v = tot_v[pl.ds(c_star * 16, 16)]
            rv = lax.rev(v, (0,))
            cs = plsc.cumsum(rv)
            m = (cs + csum_b) >= k
            i0 = 16 - jnp.sum(m.astype(jnp.int32), axis=0)
            sel = lane == i0
            cs_at = jnp.sum(jnp.where(sel, cs, 0), axis=0)
            rv_at = jnp.sum(jnp.where(sel, rv, 0), axis=0)
            j_star = c_star * 16 + 15 - i0
            kprime = k - csum_b - cs_at + rv_at

            def compact(i, cnt_v):
                for u in range(4):
                    v = data_v[pl.ds((i * 4 + u) * 16, 16)]
                    b = jnp.maximum(lax.shift_right_logical(v, 17) - HOFF, 0)
                    m = b == j_star
                    plsc.store_scatter(comp_v, [cnt_v * 16 + lane], v, mask=m)
                    cnt_v = cnt_v + m.astype(jnp.int32)
                return cnt_v

            cnt_v = lax.fori_loop(0, nch // 4, compact,
                                  jnp.zeros((16,), jnp.int32))
            maxc = jnp.max(cnt_v, axis=0)

            def count_ge(t):
                def cbody(j, acc):
                    v = comp_v[pl.ds(j * 16, 16)]
                    hit = jnp.logical_and(cnt_v > j, v >= t)
                    return acc + jnp.where(hit, 1, 0)

                accv = lax.fori_loop(0, maxc, cbody,
                                     jnp.zeros((16,), jnp.int32))
                return jnp.sum(accv, axis=0)

            def bsearch(i, carry):
                lo, hi = carry
                mid = (lo + hi) >> 1
                ge = count_ge(mid) >= kprime
                return jnp.where(ge, mid, lo), jnp.where(ge, hi, mid)

            lo0 = (j_star + HOFF) << 17
            lo, _ = lax.fori_loop(0, 17, bsearch, (lo0, lo0 + (1 << 17)))
            cnt = (k - kprime) + count_ge(lo)

            res = jnp.where(lane == 0, lo, jnp.where(lane == 1, cnt, 0))
            tot_v[pl.ds(0, 16)] = res
            pltpu.sync_copy(tot_v.at[pl.ds(0, 16)],
                            out_hbm.at[pl.ds(wid * 16, 16)])

    return body


def _make_sc_kernel(n, k_sel):
    return functools.partial(
        pl.kernel,
        out_type=jax.ShapeDtypeStruct((128,), jnp.int32),
        mesh=plsc.VectorSubcoreMesh(core_axis_name="c", subcore_axis_name="s",
                                    num_cores=2, num_subcores=16),
        compiler_params=pltpu.CompilerParams(needs_layout_passes=False),
        scratch_types=[
            pltpu.VMEM((n,), jnp.int32),
            pltpu.VMEM((16 * HB,), jnp.int32),
            pltpu.VMEM((HB,), jnp.int32),
            pltpu.VMEM((n,), jnp.int32),
        ],
    )(_make_sc_body(n, k_sel))


_sc_thresh0 = _make_sc_kernel(N0, K0)
_sc_thresh1 = _make_sc_kernel(N1, K1)


def _mul_body0(thr_ref, cnt0_ref, cnt1_ref, imp_ref, a_ref, b_ref,
               oa_ref, ob_ref, rates_ref):
    t = thr_ref[0, 0, 0]
    m = imp_ref[...] >= t
    oa_ref[...] = jnp.where(m, a_ref[...], 0.0)
    ob_ref[...] = jnp.where(m, b_ref[...], 0.0)
    c0 = cnt0_ref[0, 0]
    c1 = cnt1_ref[0, 0]
    for i in range(1, 8):
        c0 = c0 + cnt0_ref[0, i]
        c1 = c1 + cnt1_ref[0, i]
    r0 = c0.astype(jnp.float32) / (B * N0)
    r1 = c1.astype(jnp.float32) / (B * N1)
    lane2 = lax.broadcasted_iota(jnp.int32, (1, 2), 1)
    rates_ref[...] = jnp.where(lane2 == 0, r0, r1)


def _mul_body1(thr_ref, imp_ref, a_ref, b_ref, oa_ref, ob_ref):
    t = thr_ref[0, 0, 0]
    m = imp_ref[...] >= t
    oa_ref[...] = jnp.where(m, a_ref[...], 0.0)
    ob_ref[...] = jnp.where(m, b_ref[...], 0.0)


def kernel(spikes_s0_t0, spikes_s0_t1, spikes_s1_t0, spikes_s1_t1,
           imp_s0, imp_s1, training):
    bits0 = jax.lax.bitcast_convert_type(imp_s0.reshape(B, N0), jnp.int32)
    bits1 = jax.lax.bitcast_convert_type(imp_s1.reshape(B, N1), jnp.int32)

    sc1 = _sc_thresh1(bits1).reshape(8, 16)
    sc0 = _sc_thresh0(bits0).reshape(8, 16)
    thr0 = jax.lax.bitcast_convert_type(sc0[:, 0],
                                        jnp.float32).reshape(B, 1, 1)
    thr1 = jax.lax.bitcast_convert_type(sc1[:, 0],
                                        jnp.float32).reshape(B, 1, 1)
    cnt0 = sc0[:, 1].reshape(1, 8)
    cnt1 = sc1[:, 1].reshape(1, 8)

    s10 = spikes_s1_t0.reshape(B, 128, 32, 128)
    s11 = spikes_s1_t1.reshape(B, 128, 32, 128)
    imp1 = imp_s1.reshape(B, 1, 32, 128)

    spec_thr = pl.BlockSpec((1, 1, 1), lambda i: (i, 0, 0),
                            memory_space=pltpu.SMEM)
    spec_s1 = pl.BlockSpec((1, 128, 32, 128), lambda i: (i, 0, 0, 0))
    m10, m11 = pl.pallas_call(
        _mul_body1,
        grid=(B,),
        in_specs=[
            spec_thr,
            pl.BlockSpec((1, 1, 32, 128), lambda i: (i, 0, 0, 0)),
            spec_s1,
            spec_s1,
        ],
        out_specs=[spec_s1, spec_s1],
        out_shape=[
            jax.ShapeDtypeStruct(s10.shape, jnp.float32),
            jax.ShapeDtypeStruct(s11.shape, jnp.float32),
        ],
    )(thr1, imp1, s10, s11)
    m10 = m10.reshape(spikes_s1_t0.shape)
    m11 = m11.reshape(spikes_s1_t1.shape)

    spec_cnt = pl.BlockSpec((1, 8), lambda i: (0, 0),
                            memory_space=pltpu.SMEM)
    spec_s0 = pl.BlockSpec((1, 64, 128, 128), lambda i: (i, 0, 0, 0))
    m00, m01, rates = pl.pallas_call(
        _mul_body0,
        grid=(B,),
        in_specs=[
            spec_thr,
            spec_cnt,
            spec_cnt,
            pl.BlockSpec((1, 1, 128, 128), lambda i: (i, 0, 0, 0)),
            spec_s0,
            spec_s0,
        ],
        out_specs=[spec_s0, spec_s0,
                   pl.BlockSpec((1, 2), lambda i: (0, 0))],
        out_shape=[
            jax.ShapeDtypeStruct(spikes_s0_t0.shape, jnp.float32),
            jax.ShapeDtypeStruct(spikes_s0_t1.shape, jnp.float32),
            jax.ShapeDtypeStruct((1, 2), jnp.float32),
        ],
    )(thr0, cnt0, cnt1, imp_s0, spikes_s0_t0, spikes_s0_t1)

    return (m00, m01, m10, m11, rates.reshape(2))


# speedup vs baseline: 1.0153x; 1.0153x over previous
import functools

import jax
import jax.numpy as jnp
from jax import lax
from jax.experimental import pallas as pl
from jax.experimental.pallas import tpu as pltpu
from jax.experimental.pallas import tpu_sc as plsc

B = 8
N0 = 128 * 128
N1 = 64 * 64
K0 = int(0.75 * N0)
K1 = int(max(0.75, 0.9) * N1)

HB = 1024
HOFF = 7168
NCH = N0 // 16


def _sc_thresh_body(bits0_hbm, bits1_hbm, out_hbm, data_v, hist_v, tot_v,
                    comp_v):
    core = lax.axis_index("c")
    sub = lax.axis_index("s")
    wid = sub * 2 + core

    @pl.when(wid < 8)
    def _():
        pltpu.sync_copy(bits0_hbm.at[wid], data_v)

    @pl.when(jnp.logical_and(wid >= 8, wid < 16))
    def _():
        pltpu.sync_copy(bits1_hbm.at[wid - 8], data_v.at[pl.ds(0, N1)])

    @pl.when(wid < 16)
    def _():
        lane = jnp.arange(16, dtype=jnp.int32)
        ones = jnp.ones((16,), jnp.int32)
        zeros16 = jnp.zeros((16,), jnp.int32)
        lane_hb_off = lane * HB - HOFF
        k = jnp.where(wid < 8, K0, K1)

        @pl.when(wid >= 8)
        def _():
            def ztail(i, c):
                for u in range(8):
                    data_v[pl.ds(N1 + i * 128 + u * 16, 16)] = zeros16
                return c

            lax.fori_loop(0, (N0 - N1) // 128, ztail, 0)

        def zero_hist(i, c):
            for u in range(8):
                hist_v[pl.ds(i * 128 + u * 16, 16)] = zeros16
            return c

        lax.fori_loop(0, (16 * HB) // 128, zero_hist, 0)

        def hpass(i, c):
            for u in range(8):
                v = data_v[pl.ds((i * 8 + u) * 16, 16)]
                b = jnp.maximum(lax.shift_right_logical(v, 17), HOFF)
                plsc.addupdate_scatter(hist_v, [lane_hb_off + b], ones)
            return c

        lax.fori_loop(0, NCH // 8, hpass, 0)

        def collapse(i, c):
            acc = hist_v[pl.ds(i * 16, 16)]
            for l in range(1, 16):
                acc = acc + hist_v[pl.ds(l * HB + i * 16, 16)]
            tot_v[pl.ds(i * 16, 16)] = acc
            return c

        lax.fori_loop(0, HB // 16, collapse, 0)

        def scan1(i, carry):
            csum, c_star, csum_b = carry
            c = HB // 16 - 1 - i
            v = tot_v[pl.ds(c * 16, 16)]
            s = jnp.sum(v, axis=0)
            found = jnp.logical_and(csum < k, csum + s >= k)
            c_star = jnp.where(found, c, c_star)
            csum_b = jnp.where(found, csum, csum_b)
            return csum + s, c_star, csum_b

        _, c_star, csum_b = lax.fori_loop(0, HB // 16, scan1, (0, 0, 0))

        v = tot_v[pl.ds(c_star * 16, 16)]
        rv = lax.rev(v, (0,))
        cs = plsc.cumsum(rv)
        m = (cs + csum_b) >= k
        i0 = 16 - jnp.sum(m.astype(jnp.int32), axis=0)
        sel = lane == i0
        cs_at = jnp.sum(jnp.where(sel, cs, 0), axis=0)
        rv_at = jnp.sum(jnp.where(sel, rv, 0), axis=0)
        j_star = c_star * 16 + 15 - i0
        kprime = k - csum_b - cs_at + rv_at
        jsh = j_star + HOFF

        def compact(i, cnt_v):
            for u in range(8):
                v = data_v[pl.ds((i * 8 + u) * 16, 16)]
                m = lax.shift_right_logical(v, 17) == jsh
                plsc.store_scatter(comp_v, [cnt_v * 16 + lane], v, mask=m)
                cnt_v = cnt_v + m.astype(jnp.int32)
            return cnt_v

        cnt_v = lax.fori_loop(0, NCH // 8, compact,
                              jnp.zeros((16,), jnp.int32))
        maxc = jnp.max(cnt_v, axis=0)

        def count_ge(t):
            def cbody(j, acc):
                v = comp_v[pl.ds(j * 16, 16)]
                hit = jnp.logical_and(cnt_v > j, v >= t)
                return acc + jnp.where(hit, 1, 0)

            accv = lax.fori_loop(0, maxc, cbody,
                                 jnp.zeros((16,), jnp.int32))
            return jnp.sum(accv, axis=0)

        def bsearch(i, carry):
            lo, hi = carry
            mid = (lo + hi) >> 1
            ge = count_ge(mid) >= kprime
            return jnp.where(ge, mid, lo), jnp.where(ge, hi, mid)

        lo0 = jsh << 17
        lo, _ = lax.fori_loop(0, 17, bsearch, (lo0, lo0 + (1 << 17)))
        cnt = (k - kprime) + count_ge(lo)

        res = jnp.where(lane == 0, lo, jnp.where(lane == 1, cnt, 0))
        tot_v[pl.ds(0, 16)] = res
        pltpu.sync_copy(tot_v.at[pl.ds(0, 16)],
                        out_hbm.at[pl.ds(wid * 16, 16)])


_sc_thresh = functools.partial(
    pl.kernel,
    out_type=jax.ShapeDtypeStruct((256,), jnp.int32),
    mesh=plsc.VectorSubcoreMesh(core_axis_name="c", subcore_axis_name="s",
                                num_cores=2, num_subcores=16),
    compiler_params=pltpu.CompilerParams(needs_layout_passes=False),
    scratch_types=[
        pltpu.VMEM((N0,), jnp.int32),
        pltpu.VMEM((16 * HB,), jnp.int32),
        pltpu.VMEM((HB,), jnp.int32),
        pltpu.VMEM((N0,), jnp.int32),
    ],
)(_sc_thresh_body)


def _mul_body(thr0_ref, thr1_ref, cnt_ref, imp0_ref, imp1_ref,
              a0_ref, b0_ref, a1_ref, b1_ref,
              oa0_ref, ob0_ref, oa1_ref, ob1_ref, rates_ref):
    t0 = thr0_ref[0, 0, 0]
    m0 = imp0_ref[...] >= t0
    oa0_ref[...] = jnp.where(m0, a0_ref[...], 0.0)
    ob0_ref[...] = jnp.where(m0, b0_ref[...], 0.0)
    t1 = thr1_ref[0, 0, 0]
    m1 = imp1_ref[...] >= t1
    oa1_ref[...] = jnp.where(m1, a1_ref[...], 0.0)
    ob1_ref[...] = jnp.where(m1, b1_ref[...], 0.0)
    c0 = cnt_ref[0, 0]
    c1 = cnt_ref[0, 8]
    for i in range(1, 8):
        c0 = c0 + cnt_ref[0, i]
        c1 = c1 + cnt_ref[0, 8 + i]
    r0 = c0.astype(jnp.float32) / (B * N0)
    r1 = c1.astype(jnp.float32) / (B * N1)
    lane2 = lax.broadcasted_iota(jnp.int32, (1, 2), 1)
    rates_ref[...] = jnp.where(lane2 == 0, r0, r1)


def kernel(spikes_s0_t0, spikes_s0_t1, spikes_s1_t0, spikes_s1_t1,
           imp_s0, imp_s1, training):
    bits0 = jax.lax.bitcast_convert_type(imp_s0.reshape(B, N0), jnp.int32)
    bits1 = jax.lax.bitcast_convert_type(imp_s1.reshape(B, N1), jnp.int32)

    sc_out = _sc_thresh(bits0, bits1).reshape(16, 16)
    thr_bits = sc_out[:, 0]
    counts = sc_out[:, 1].reshape(1, 16)
    thr0 = jax.lax.bitcast_convert_type(thr_bits[0:8],
                                        jnp.float32).reshape(B, 1, 1)
    thr1 = jax.lax.bitcast_convert_type(thr_bits[8:16],
                                        jnp.float32).reshape(B, 1, 1)

    s10 = spikes_s1_t0.reshape(B, 128, 32, 128)
    s11 = spikes_s1_t1.reshape(B, 128, 32, 128)
    imp1 = imp_s1.reshape(B, 1, 32, 128)

    spec_thr = pl.BlockSpec((1, 1, 1), lambda i, j: (i, 0, 0),
                            memory_space=pltpu.SMEM)
    spec_s0 = pl.BlockSpec((1, 32, 128, 128), lambda i, j: (i, j, 0, 0))
    spec_s1 = pl.BlockSpec((1, 64, 32, 128), lambda i, j: (i, j, 0, 0))
    m00, m01, m10, m11, rates = pl.pallas_call(
        _mul_body,
        grid=(B, 2),
        in_specs=[
            spec_thr,
            spec_thr,
            pl.BlockSpec((1, 16), lambda i, j: (0, 0),
                         memory_space=pltpu.SMEM),
            pl.BlockSpec((1, 1, 128, 128), lambda i, j: (i, 0, 0, 0)),
            pl.BlockSpec((1, 1, 32, 128), lambda i, j: (i, 0, 0, 0)),
            spec_s0,
            spec_s0,
            spec_s1,
            spec_s1,
        ],
        out_specs=[spec_s0, spec_s0, spec_s1, spec_s1,
                   pl.BlockSpec((1, 2), lambda i, j: (0, 0))],
        out_shape=[
            jax.ShapeDtypeStruct(spikes_s0_t0.shape, jnp.float32),
            jax.ShapeDtypeStruct(spikes_s0_t1.shape, jnp.float32),
            jax.ShapeDtypeStruct(s10.shape, jnp.float32),
            jax.ShapeDtypeStruct(s11.shape, jnp.float32),
            jax.ShapeDtypeStruct((1, 2), jnp.float32),
        ],
    )(thr0, thr1, counts, imp_s0, imp1,
      spikes_s0_t0, spikes_s0_t1, s10, s11)

    m10 = m10.reshape(spikes_s1_t0.shape)
    m11 = m11.reshape(spikes_s1_t1.shape)
    return (m00, m01, m10, m11, rates.reshape(2))
